# baseline (speedup 1.0000x reference)
import jax
import jax.numpy as jnp
from jax.experimental import pallas as pl
from jax.experimental.pallas import tpu as pltpu

_G0, _G1, _D = 32, 32, 256
_N = _G0 * _G1
_B = 512
_LR = 0.1
_SIGMA = 2.0


def _som_body(gT_ref, imgs_ref, w_ref, out_ref, sq_ref, ip_ref):
    out_ref[:, :] = gT_ref[:, :]
    lane_iota = jax.lax.broadcasted_iota(jnp.int32, (1, _N), 1)

    ip_ref[:, 0:1] = imgs_ref[pl.ds(0, 1), :].T
    for c in range(_D // 8):
        sl = pl.ds(8 * c, 8)
        ic = ip_ref[sl, 0:1]
        d0 = out_ref[sl, :] - ic
        sq_ref[sl, :] = d0 * d0

    def step(t, carry):
        d2 = jnp.sum(sq_ref[:, :], axis=0, keepdims=True)
        m = jnp.min(d2)
        k = jnp.min(jnp.where(d2 == m, lane_iota, _N))
        a = w_ref[pl.ds(k, 1), :]
        ip_ref[:, 0:1] = imgs_ref[pl.ds(t, 1), :].T
        ip_ref[:, 1:2] = imgs_ref[pl.ds(t + 1, 1), :].T

        def chunk(c, _):
            sl = pl.ds(8 * c, 8)
            ip = ip_ref[sl, :]
            ic = ip[:, 0:1]
            inx = ip[:, 1:2]
            g = out_ref[sl, :]
            diff = g - ic
            gp = g - a * diff
            out_ref[sl, :] = gp
            dn = gp - inx
            sq_ref[sl, :] = dn * dn
            return _

        jax.lax.fori_loop(0, _D // 8, chunk, 0, unroll=2)
        return carry

    jax.lax.fori_loop(0, _B, step, 0, unroll=2)


def kernel(grade, imgs):
    gT = grade.reshape(_N, _D).T
    imgs_pad = jnp.concatenate([imgs, imgs[-1:, :]], axis=0)
    k1 = jnp.arange(_N, dtype=jnp.int32)
    i1 = (k1 // _G1).astype(jnp.float32)
    j1 = (k1 % _G1).astype(jnp.float32)
    di = i1[:, None] - i1[None, :]
    dj = j1[:, None] - j1[None, :]
    d = jnp.sqrt(di * di + dj * dj)
    w = jnp.float32(_LR) * jnp.exp(-(d * d) / (2.0 * jnp.float32(_SIGMA) ** 2))
    outT = pl.pallas_call(
        _som_body,
        out_shape=jax.ShapeDtypeStruct((_D, _N), jnp.float32),
        scratch_shapes=[
            pltpu.VMEM((_D, _N), jnp.float32),
            pltpu.VMEM((_D, 2), jnp.float32),
        ],
    )(gT, imgs_pad, w)
    return outT.T.reshape(_G0, _G1, _D)


# speedup vs baseline: 2.4739x; 2.4739x over previous
import jax
import jax.numpy as jnp
from jax.experimental import pallas as pl
from jax.experimental.pallas import tpu as pltpu

_G0, _G1, _D = 32, 32, 256
_N = _G0 * _G1
_B = 512
_LR = 0.1
_SIGMA = 2.0


def _som_body(gT_ref, imgs_ref, w_ref, out_ref, ip_ref):
    out_ref[:, :] = gT_ref[:, :]
    lane_iota = jax.lax.broadcasted_iota(jnp.int32, (1, _N), 1)

    ip_ref[:, 0:1] = imgs_ref[pl.ds(0, 1), :].T
    acc0 = jnp.zeros((8, _N), jnp.float32)
    for c in range(_D // 8):
        sl = pl.ds(8 * c, 8)
        ic = ip_ref[sl, 0:1]
        d0 = out_ref[sl, :] - ic
        acc0 = acc0 + d0 * d0
    d2_0 = jnp.sum(acc0, axis=0, keepdims=True)

    def step(t, d2):
        m = jnp.min(d2)
        k = jnp.min(jnp.where(d2 == m, lane_iota, _N))
        a = w_ref[pl.ds(k, 1), :]
        ip_ref[:, 0:1] = imgs_ref[pl.ds(t, 1), :].T
        ip_ref[:, 1:2] = imgs_ref[pl.ds(t + 1, 1), :].T

        acc = jnp.zeros((8, _N), jnp.float32)
        for c in range(_D // 8):
            sl = pl.ds(8 * c, 8)
            ip = ip_ref[sl, :]
            ic = ip[:, 0:1]
            inx = ip[:, 1:2]
            g = out_ref[sl, :]
            diff = g - ic
            gp = g - a * diff
            out_ref[sl, :] = gp
            dn = gp - inx
            acc = acc + dn * dn
        return jnp.sum(acc, axis=0, keepdims=True)

    jax.lax.fori_loop(0, _B, step, d2_0, unroll=2)


def kernel(grade, imgs):
    gT = grade.reshape(_N, _D).T
    imgs_pad = jnp.concatenate([imgs, imgs[-1:, :]], axis=0)
    k1 = jnp.arange(_N, dtype=jnp.int32)
    i1 = (k1 // _G1).astype(jnp.float32)
    j1 = (k1 % _G1).astype(jnp.float32)
    di = i1[:, None] - i1[None, :]
    dj = j1[:, None] - j1[None, :]
    d = jnp.sqrt(di * di + dj * dj)
    w = jnp.float32(_LR) * jnp.exp(-(d * d) / (2.0 * jnp.float32(_SIGMA) ** 2))
    outT = pl.pallas_call(
        _som_body,
        out_shape=jax.ShapeDtypeStruct((_D, _N), jnp.float32),
        scratch_shapes=[
            pltpu.VMEM((_D, 2), jnp.float32),
        ],
    )(gT, imgs_pad, w)
    return outT.T.reshape(_G0, _G1, _D)


# speedup vs baseline: 2.7370x; 1.1064x over previous
import jax
import jax.numpy as jnp
from jax.experimental import pallas as pl

_G0, _G1, _D = 32, 32, 256
_N = _G0 * _G1
_B = 512
_LR = 0.1
_SIGMA = 2.0


def _som_body(gT_ref, imgs_ref, w_ref, out_ref):
    out_ref[:, :] = gT_ref[:, :]
    lane_iota = jax.lax.broadcasted_iota(jnp.int32, (1, _N), 1)

    img0 = imgs_ref[pl.ds(0, 1), :].T
    diff0 = out_ref[:, :] - img0
    d2_0 = jnp.sum(diff0 * diff0, axis=0, keepdims=True)

    def step(t, d2):
        m = jnp.min(d2)
        k = jnp.min(jnp.where(d2 == m, lane_iota, _N))
        a = w_ref[pl.ds(k, 1), :]
        img = imgs_ref[pl.ds(t, 1), :].T
        g = out_ref[:, :]
        diff = g - img
        gp = g - a * diff
        out_ref[:, :] = gp
        imgn = imgs_ref[pl.ds(t + 1, 1), :].T
        diffn = gp - imgn
        return jnp.sum(diffn * diffn, axis=0, keepdims=True)

    jax.lax.fori_loop(0, _B, step, d2_0, unroll=2)


def kernel(grade, imgs):
    gT = grade.reshape(_N, _D).T
    imgs_pad = jnp.concatenate([imgs, imgs[-1:, :]], axis=0)
    k1 = jnp.arange(_N, dtype=jnp.int32)
    i1 = (k1 // _G1).astype(jnp.float32)
    j1 = (k1 % _G1).astype(jnp.float32)
    di = i1[:, None] - i1[None, :]
    dj = j1[:, None] - j1[None, :]
    d = jnp.sqrt(di * di + dj * dj)
    w = jnp.float32(_LR) * jnp.exp(-(d * d) / (2.0 * jnp.float32(_SIGMA) ** 2))
    outT = pl.pallas_call(
        _som_body,
        out_shape=jax.ShapeDtypeStruct((_D, _N), jnp.float32),
    )(gT, imgs_pad, w)
    return outT.T.reshape(_G0, _G1, _D)
